# baseline (speedup 1.0000x reference)
import functools

import jax
import jax.numpy as jnp
from jax import lax
from jax.experimental import pallas as pl
from jax.experimental.pallas import tpu as pltpu
from jax.experimental.pallas import tpu_sc as plsc

B = 4096
D = 32
NC = 2
NS = 16
NW = NC * NS
B_PER_W = B // NW
L = 16
NBUF = 8


def _sc_gather(user_emb_t, item_emb_t, user_idx, item_idx):
    mesh = plsc.VectorSubcoreMesh(core_axis_name="c", subcore_axis_name="s")

    @functools.partial(
        pl.kernel,
        mesh=mesh,
        compiler_params=pltpu.CompilerParams(needs_layout_passes=False),
        out_type=(
            jax.ShapeDtypeStruct((B, D), jnp.float32),
            jax.ShapeDtypeStruct((B, D), jnp.float32),
        ),
        scratch_types=[
            pltpu.VMEM((B_PER_W,), jnp.int32),
            pltpu.VMEM((B_PER_W,), jnp.int32),
            pltpu.VMEM((B_PER_W, D), jnp.float32),
            pltpu.VMEM((B_PER_W, D), jnp.float32),
            pltpu.VMEM((NBUF, D, 128), jnp.float32),
            pltpu.VMEM((NBUF, D, 128), jnp.float32),
            pltpu.SemaphoreType.DMA,
            pltpu.SemaphoreType.DMA,
        ],
    )
    def gather_kernel(uemb, iemb, uidx, iidx, wu_out, hi_out,
                      uidx_v, iidx_v, urows, irows, uwin, iwin, usem, isem):
        wid = lax.axis_index("s") * NC + lax.axis_index("c")
        base = wid * B_PER_W
        pltpu.sync_copy(uidx.at[pl.ds(base, B_PER_W)], uidx_v)
        pltpu.sync_copy(iidx.at[pl.ds(base, B_PER_W)], iidx_v)
        lanes = lax.iota(jnp.int32, L)
        rlo = lax.iota(jnp.int32, L)
        rhi = rlo + L

        def scalar_at(vec, l):
            return lax.reduce_max(jnp.where(lanes == l, vec, 0), (0,))

        def chunk(j):
            uvec = uidx_v[pl.ds(j * L, L)]
            ivec = iidx_v[pl.ds(j * L, L)]
            for half in range(2):
                copies = []
                for s in range(NBUF):
                    l = half * NBUF + s
                    ur = scalar_at(uvec, l)
                    ir = scalar_at(ivec, l)
                    uw = pl.multiple_of((ur >> 7) << 7, 128)
                    iw = pl.multiple_of((ir >> 7) << 7, 128)
                    copies.append(pltpu.async_copy(
                        uemb.at[:, pl.ds(uw, 128)], uwin.at[s], usem))
                    copies.append(pltpu.async_copy(
                        iemb.at[:, pl.ds(iw, 128)], iwin.at[s], isem))
                for c in copies:
                    c.wait()
                for s in range(NBUF):
                    l = half * NBUF + s
                    i = j * L + l
                    uc = lax.broadcast(scalar_at(uvec, l) & 127, (L,))
                    ic = lax.broadcast(scalar_at(ivec, l) & 127, (L,))
                    urows[i, pl.ds(0, L)] = plsc.load_gather(
                        uwin.at[s], [rlo, uc])
                    urows[i, pl.ds(L, L)] = plsc.load_gather(
                        uwin.at[s], [rhi, uc])
                    irows[i, pl.ds(0, L)] = plsc.load_gather(
                        iwin.at[s], [rlo, ic])
                    irows[i, pl.ds(L, L)] = plsc.load_gather(
                        iwin.at[s], [rhi, ic])

        pl.loop(0, B_PER_W // L)(chunk)
        pltpu.sync_copy(urows, wu_out.at[pl.ds(base, B_PER_W), :])
        pltpu.sync_copy(irows, hi_out.at[pl.ds(base, B_PER_W), :])

    return gather_kernel(user_emb_t, item_emb_t, user_idx, item_idx)


BM = 512


def _mm_body(w_ref, h_ref, o_ref):
    o_ref[...] = lax.dot_general(
        w_ref[...], h_ref[...],
        (((1,), (1,)), ((), ())),
        preferred_element_type=jnp.float32,
    )


def _tc_matmul(w_u, h_i):
    return pl.pallas_call(
        _mm_body,
        grid=(B // BM,),
        in_specs=[
            pl.BlockSpec((BM, D), lambda i: (i, 0)),
            pl.BlockSpec((B, D), lambda i: (0, 0)),
        ],
        out_specs=pl.BlockSpec((BM, B), lambda i: (i, 0)),
        out_shape=jax.ShapeDtypeStruct((B, B), jnp.float32),
    )(w_u, h_i)


def kernel(user_tensor, item_tensor, user_emb, item_emb):
    w_u, h_i = _sc_gather(user_emb.T, item_emb.T, user_tensor, item_tensor)
    return _tc_matmul(w_u, h_i)
